# baseline (speedup 1.0000x reference)
import functools

import jax
import jax.numpy as jnp
from jax import lax
from jax.experimental import pallas as pl
from jax.experimental.pallas import tpu as pltpu
from jax.experimental.pallas import tpu_sc as plsc

BATCH = 16384
DIM = 64
NC = 2
NS = 16
L = 16
NW = NC * NS
BPW = BATCH // NW

_mesh = plsc.VectorSubcoreMesh(core_axis_name="c", subcore_axis_name="s")


@functools.partial(
    pl.kernel,
    mesh=_mesh,
    compiler_params=pltpu.CompilerParams(use_tc_tiling_on_sc=False),
    out_type=jax.ShapeDtypeStruct((BATCH, DIM), jnp.float32),
    scratch_types=[
        pltpu.VMEM((BPW,), jnp.int32),
        pltpu.VMEM((BPW,), jnp.int32),
        pltpu.VMEM((BPW, DIM), jnp.float32),
        pltpu.VMEM((BPW, DIM), jnp.float32),
        pltpu.SemaphoreType.DMA,
        pltpu.SemaphoreType.DMA,
    ],
)
def _distmult_sc(e1_hbm, r_hbm, ent_hbm, rel_hbm, out_hbm,
                 e_idx, r_idx, e_rows, r_rows, sem_e, sem_r):
    wid = lax.axis_index("s") * NC + lax.axis_index("c")
    base = wid * BPW
    pltpu.sync_copy(e1_hbm.at[pl.ds(base, BPW)], e_idx)
    pltpu.sync_copy(r_hbm.at[pl.ds(base, BPW)], r_idx)
    ce = pltpu.async_copy(ent_hbm.at[e_idx], e_rows, sem_e)
    cr = pltpu.async_copy(rel_hbm.at[r_idx], r_rows, sem_r)
    ce.wait()
    cr.wait()

    def body(i, _):
        for j in range(DIM // L):
            sl = pl.ds(j * L, L)
            e_rows[i, sl] = e_rows[i, sl] * r_rows[i, sl]
        return ()

    lax.fori_loop(0, BPW, body, ())
    pltpu.sync_copy(e_rows, out_hbm.at[pl.ds(base, BPW)])


def kernel(e1, r, entity_table, relation_table):
    return _distmult_sc(e1.astype(jnp.int32), r.astype(jnp.int32),
                        entity_table, relation_table)


# speedup vs baseline: 1.6940x; 1.6940x over previous
import functools

import jax
import jax.numpy as jnp
from jax import lax
from jax.experimental import pallas as pl
from jax.experimental.pallas import tpu as pltpu
from jax.experimental.pallas import tpu_sc as plsc

BATCH = 16384
DIM = 64
NC = 2
NS = 16
L = 16
NW = NC * NS
BPW = BATCH // NW
CH = 256
NCH = BPW // CH

_mesh = plsc.VectorSubcoreMesh(core_axis_name="c", subcore_axis_name="s")


@functools.partial(
    pl.kernel,
    mesh=_mesh,
    out_type=jax.ShapeDtypeStruct((BATCH, DIM), jnp.float32),
    scratch_types=[
        pltpu.VMEM_SHARED((NW, 2, BPW), jnp.int32),
        pltpu.SMEM((BPW,), jnp.int32),
        pltpu.SMEM((BPW,), jnp.int32),
        pltpu.VMEM((CH, DIM), jnp.float32),
        pltpu.VMEM((CH, DIM), jnp.float32),
        pltpu.SemaphoreType.DMA,
        pltpu.SemaphoreType.DMA,
    ],
)
def _distmult_sc(e1_hbm, r_hbm, ent_hbm, rel_hbm, out_hbm,
                 idx_sh, e_idx, r_idx, e_rows, r_rows, sem_e, sem_r):
    wid = lax.axis_index("s") * NC + lax.axis_index("c")
    base = wid * BPW
    pltpu.sync_copy(e1_hbm.at[pl.ds(base, BPW)], idx_sh.at[wid, 0])
    pltpu.sync_copy(r_hbm.at[pl.ds(base, BPW)], idx_sh.at[wid, 1])
    pltpu.sync_copy(idx_sh.at[wid, 0], e_idx)
    pltpu.sync_copy(idx_sh.at[wid, 1], r_idx)

    for c in range(NCH):
        def fire(i, _):
            pltpu.async_copy(ent_hbm.at[e_idx[c * CH + i]], e_rows.at[i], sem_e)
            pltpu.async_copy(rel_hbm.at[r_idx[c * CH + i]], r_rows.at[i], sem_r)
            return ()

        lax.fori_loop(0, CH, fire, ())

        def drain(i, _):
            pltpu.make_async_copy(ent_hbm.at[0], e_rows.at[0], sem_e).wait()
            pltpu.make_async_copy(rel_hbm.at[0], r_rows.at[0], sem_r).wait()
            return ()

        lax.fori_loop(0, CH, drain, ())

        def mult(i, _):
            for j in range(DIM // L):
                sl = pl.ds(j * L, L)
                e_rows[i, sl] = e_rows[i, sl] * r_rows[i, sl]
            return ()

        lax.fori_loop(0, CH, mult, ())
        pltpu.sync_copy(e_rows, out_hbm.at[pl.ds(base + c * CH, CH)])


def kernel(e1, r, entity_table, relation_table):
    return _distmult_sc(e1.astype(jnp.int32), r.astype(jnp.int32),
                        entity_table, relation_table)
